# Initial kernel scaffold; baseline (speedup 1.0000x reference)
import jax
import jax.numpy as jnp
from jax.experimental import pallas as pl


def kernel(x, edge_index, edge_attr, batch, embed_W, embed_b, edge_W, edge_b, msg_W, msg_b, upd_W, upd_b, pred_W, pred_b):
    raise NotImplementedError("write your pallas kernel here")


# speedup vs baseline: 5.4526x; 5.4526x over previous
import functools
import jax
import jax.numpy as jnp
from jax import lax
from jax.experimental import pallas as pl
from jax.experimental.pallas import tpu as pltpu
from jax.experimental.pallas import tpu_sc as plsc

_N = 10000
_E = 320000
_H = 128
_G = 64
_L = 3

_NC = 2
_NS = 16
_NW = _NC * _NS
_CH = 128
_NCHUNK = 80
_EW = _NCHUNK * _CH
_EP = _NW * _EW
_NA = 10240
_RPT = _NA // _NS

_BLK = 1000
_GRID = _N // _BLK

_f32 = jnp.float32
_sc_mesh = plsc.VectorSubcoreMesh(core_axis_name="c", subcore_axis_name="s")


@functools.partial(
    pl.kernel,
    out_type=jax.ShapeDtypeStruct((_NC, _NA, _H), _f32),
    mesh=_sc_mesh,
    scratch_types=[
        pltpu.VMEM((_NCHUNK, _CH), jnp.int32),
        pltpu.VMEM((_NCHUNK, _CH), jnp.int32),
        pltpu.VMEM((_CH, _H), _f32),
        pltpu.VMEM_SHARED((_NA, _H), _f32),
        pltpu.SemaphoreType.DMA,
    ],
)
def _sc_propagate(A_hbm, send_hbm, rec_hbm, z_hbm, out_hbm,
                  idx_s, idx_r, gbuf, acc, sem):
    c = lax.axis_index("c")
    sid = lax.axis_index("s")
    w = sid * _NC + c
    pltpu.sync_copy(send_hbm.at[w], idx_s)
    pltpu.sync_copy(rec_hbm.at[w], idx_r)
    pltpu.sync_copy(z_hbm, acc.at[pl.ds(sid * _RPT, _RPT)])
    plsc.subcore_barrier()

    @pl.loop(0, _NCHUNK)
    def _chunk(j):
        pltpu.async_copy(A_hbm.at[idx_s.at[j]], gbuf, sem).wait()
        pltpu.sync_copy(gbuf, acc.at[idx_r.at[j]], add=True)

    plsc.subcore_barrier()
    pltpu.sync_copy(acc.at[pl.ds(sid * _RPT, _RPT)],
                    out_hbm.at[c, pl.ds(sid * _RPT, _RPT)])


_SW = 8


@functools.partial(
    pl.kernel,
    out_type=jax.ShapeDtypeStruct((_NC, _NA, _SW), _f32),
    mesh=_sc_mesh,
    scratch_types=[
        pltpu.VMEM((_NCHUNK, _CH), jnp.int32),
        pltpu.VMEM((_CH, _SW), _f32),
        pltpu.VMEM_SHARED((_NA, _SW), _f32),
        pltpu.SemaphoreType.DMA,
    ],
)
def _sc_edge_stats(rec_hbm, ev_hbm, zs_hbm, out_hbm, idx_r, ebuf, acc, sem):
    c = lax.axis_index("c")
    sid = lax.axis_index("s")
    w = sid * _NC + c
    pltpu.sync_copy(rec_hbm.at[w], idx_r)
    pltpu.sync_copy(zs_hbm, acc.at[pl.ds(sid * _RPT, _RPT)])
    plsc.subcore_barrier()

    @pl.loop(0, _NCHUNK)
    def _chunk(j):
        pltpu.sync_copy(ev_hbm.at[w, j], ebuf)
        pltpu.sync_copy(ebuf, acc.at[idx_r.at[j]], add=True)

    plsc.subcore_barrier()
    pltpu.sync_copy(acc.at[pl.ds(sid * _RPT, _RPT)],
                    out_hbm.at[c, pl.ds(sid * _RPT, _RPT)])


def _prep_body(msgW, msgb, updW, updb, ew, eb, w2u2_o, rows_o):
    W2 = msgW[0, _H:2 * _H, :]
    W3 = msgW[0, 2 * _H:, :]
    U2 = updW[0, _H:, :]
    u = jnp.dot(ew[...], W3, preferred_element_type=_f32)
    cc = jnp.dot(eb[...], W3, preferred_element_type=_f32) + msgb[0]
    uU2 = jnp.dot(u, U2, preferred_element_type=_f32)
    cU2 = jnp.dot(cc, U2, preferred_element_type=_f32)
    w2u2_o[...] = jnp.dot(W2, U2, preferred_element_type=_f32)[None]
    rows_o[...] = jnp.concatenate(
        [uU2, cU2, updb[0], jnp.zeros((5, _H), _f32)], axis=0)[None]


def _embed_body(x, eW, eb1, W1, h_o, A_o):
    h = jnp.dot(x[...], eW[...], preferred_element_type=_f32) + eb1[...]
    h_o[...] = h
    A_o[...] = jnp.dot(h, W1[...], preferred_element_type=_f32)


def _update_body(h, P0, P1, s0, s1, U1, U2, W2U2, rows, W1n, h_o, A_o):
    hv = h[...]
    P = P0[...] + P1[...]
    st = s0[...] + s1[...]
    deg = st[:, 0:1]
    s = st[:, 1:2]
    r = rows[...]
    t = (jnp.dot(hv, U1[...], preferred_element_type=_f32)
         + jnp.dot(P, U2[...], preferred_element_type=_f32)
         + deg * jnp.dot(hv, W2U2[...], preferred_element_type=_f32)
         + s * r[0:1, :] + deg * r[1:2, :] + r[2:3, :])
    hn = hv + jnp.maximum(t, 0.0)
    h_o[...] = hn
    A_o[...] = jnp.dot(hn, W1n[...], preferred_element_type=_f32)


def _pool_body(h, bt, pp, out_o, acc):
    k = pl.program_id(0)

    @pl.when(k == 0)
    def _init():
        acc[...] = jnp.zeros((_G, _H), _f32)

    bb = bt[...]
    m = (bb == lax.broadcasted_iota(jnp.int32, (_BLK, _G), 1)).astype(_f32)
    acc[...] += lax.dot_general(m, h[...], (((0,), (0,)), ((), ())),
                                preferred_element_type=_f32)

    @pl.when(k == _GRID - 1)
    def _fin():
        a = acc[...]
        out_o[...] = (jnp.sum(a * pp[0:1, :], axis=1, keepdims=True)
                      + pp[1:2, 0:1])


def _row_spec():
    return pl.BlockSpec((_BLK, _H), lambda i: (i, 0))


def _full_spec(shape):
    nd = len(shape)
    return pl.BlockSpec(shape, lambda i, _n=nd: (0,) * _n)


_tc_prep = pl.pallas_call(
    _prep_body,
    grid=(_L,),
    in_specs=[
        pl.BlockSpec((1, 3 * _H, _H), lambda i: (i, 0, 0)),
        pl.BlockSpec((1, 1, _H), lambda i: (i, 0, 0)),
        pl.BlockSpec((1, 2 * _H, _H), lambda i: (i, 0, 0)),
        pl.BlockSpec((1, 1, _H), lambda i: (i, 0, 0)),
        _full_spec((1, _H)),
        _full_spec((1, _H)),
    ],
    out_specs=[
        pl.BlockSpec((1, _H, _H), lambda i: (i, 0, 0)),
        pl.BlockSpec((1, 8, _H), lambda i: (i, 0, 0)),
    ],
    out_shape=[
        jax.ShapeDtypeStruct((_L, _H, _H), _f32),
        jax.ShapeDtypeStruct((_L, 8, _H), _f32),
    ],
)

_tc_embed = pl.pallas_call(
    _embed_body,
    grid=(_GRID,),
    in_specs=[
        _row_spec(),
        _full_spec((_H, _H)),
        _full_spec((1, _H)),
        _full_spec((_H, _H)),
    ],
    out_specs=[_row_spec(), _row_spec()],
    out_shape=[
        jax.ShapeDtypeStruct((_N, _H), _f32),
        jax.ShapeDtypeStruct((_N, _H), _f32),
    ],
)

_tc_update = pl.pallas_call(
    _update_body,
    grid=(_GRID,),
    in_specs=[
        _row_spec(),
        _row_spec(),
        _row_spec(),
        pl.BlockSpec((_BLK, _SW), lambda i: (i, 0)),
        pl.BlockSpec((_BLK, _SW), lambda i: (i, 0)),
        _full_spec((_H, _H)),
        _full_spec((_H, _H)),
        _full_spec((_H, _H)),
        _full_spec((8, _H)),
        _full_spec((_H, _H)),
    ],
    out_specs=[_row_spec(), _row_spec()],
    out_shape=[
        jax.ShapeDtypeStruct((_N, _H), _f32),
        jax.ShapeDtypeStruct((_N, _H), _f32),
    ],
)

_tc_pool = pl.pallas_call(
    _pool_body,
    grid=(_GRID,),
    in_specs=[
        _row_spec(),
        pl.BlockSpec((_BLK, 1), lambda i: (i, 0)),
        _full_spec((8, _H)),
    ],
    out_specs=pl.BlockSpec((_G, 1), lambda i: (0, 0)),
    out_shape=jax.ShapeDtypeStruct((_G, 1), _f32),
    scratch_shapes=[pltpu.VMEM((_G, _H), _f32)],
    compiler_params=pltpu.CompilerParams(
        dimension_semantics=("arbitrary",)),
)


def kernel(x, edge_index, edge_attr, batch, embed_W, embed_b, edge_W, edge_b,
           msg_W, msg_b, upd_W, upd_b, pred_W, pred_b):
    pad = _EP - _E
    send = edge_index[0].astype(jnp.int32)
    rec = edge_index[1].astype(jnp.int32)
    spad = jnp.arange(pad, dtype=jnp.int32) % _N
    rpad = _N + jnp.arange(pad, dtype=jnp.int32) % (_NA - _N)
    send_r = jnp.concatenate([send, spad]).reshape(_NW, _NCHUNK, _CH)
    rec_r = jnp.concatenate([rec, rpad]).reshape(_NW, _NCHUNK, _CH)
    z = jnp.zeros((_RPT, _H), _f32)

    ev = jnp.zeros((_EP, _SW), _f32).at[:, 0].set(1.0) \
                                    .at[:_E, 1].set(edge_attr.astype(_f32))
    ev_r = ev.reshape(_NW, _NCHUNK, _CH, _SW)
    zs = jnp.zeros((_RPT, _SW), _f32)
    stats = _sc_edge_stats(rec_r, ev_r, zs)

    msgb3 = msg_b.reshape(_L, 1, _H)
    updb3 = upd_b.reshape(_L, 1, _H)
    eb1 = edge_b.reshape(1, _H)
    w2u2, rows = _tc_prep(msg_W, msgb3, upd_W, updb3, edge_W, eb1)

    embb = embed_b.reshape(1, _H)
    h, A = _tc_embed(x.astype(_f32), embed_W, embb, msg_W[0, :_H, :])

    for i in range(_L):
        Pp = _sc_propagate(A, send_r, rec_r, z)
        W1n = msg_W[i + 1, :_H, :] if i + 1 < _L else msg_W[0, :_H, :]
        h, A = _tc_update(h, Pp[0], Pp[1], stats[0], stats[1],
                          upd_W[i, :_H, :], upd_W[i, _H:, :],
                          w2u2[i], rows[i], W1n)

    batch2 = batch.reshape(_N, 1).astype(jnp.int32)
    ppack = jnp.zeros((8, _H), _f32).at[0, :].set(pred_W[:, 0]) \
                                    .at[1, 0].set(pred_b[0])
    out2 = _tc_pool(h, batch2, ppack)
    return out2[:, 0]
